# baseline (speedup 1.0000x reference)
import functools

import jax
import jax.numpy as jnp
from jax import lax
from jax.experimental import pallas as pl
from jax.experimental.pallas import tpu as pltpu
from jax.experimental.pallas import tpu_sc as plsc

N_NODES = 10000
IN_CH = 128
HID = 64
LAT = 64
N_EDGES = 320000

NC, NS = 2, 16
NT = NC * NS
NPT = 320
NPAD = NT * NPT
CAP = 12288
NSEG = 4
CAPS = CAP // NSEG
CHUNK = 16000
G = 320

DEC_N = N_NODES * IN_CH
DEC_BLK = 51200
DEC_GRID = DEC_N // DEC_BLK

_mesh = plsc.VectorSubcoreMesh(core_axis_name="c", subcore_axis_name="s")


def _wid():
    return lax.axis_index("s") * NC + lax.axis_index("c")


def _preproc_body(src_hbm, dst_hbm, slist_hbm, dlist_hbm, cnt_hbm, deg_hbm,
                  csrc_a, cdst_a, csrc_b, cdst_b, slist_v, dlist_v, deg_v,
                  cntv_v, sem_a, sem_b):
    wid = _wid()
    lo = wid * NPT
    hi = lo + NPT
    zeros16i = jnp.zeros((16,), jnp.int32)
    ones16f = jnp.ones((16,), jnp.float32)

    def init_deg(i, _):
        deg_v[pl.ds(i * 16, 16)] = ones16f
        return 0
    lax.fori_loop(0, NPT // 16, init_deg, 0)

    def init_lists(i, _):
        slist_v[pl.ds(i * 16, 16)] = zeros16i
        return 0
    lax.fori_loop(0, CAP // 16, init_lists, 0)

    bufs = ((csrc_a, cdst_a, sem_a), (csrc_b, cdst_b, sem_b))

    def issue(c, b):
        s_, d_, sm = bufs[b]
        pltpu.async_copy(src_hbm.at[pl.ds(c * CHUNK, CHUNK)], s_, sm)
        pltpu.async_copy(dst_hbm.at[pl.ds(c * CHUNK, CHUNK)], d_, sm)

    def wait(b):
        s_, d_, sm = bufs[b]
        pltpu.make_async_copy(src_hbm.at[pl.ds(0, CHUNK)], s_, sm).wait()
        pltpu.make_async_copy(src_hbm.at[pl.ds(0, CHUNK)], d_, sm).wait()

    def scan_chunk(b, cnt):
        s_, d_, sm = bufs[b]

        def grp(gi, cnt):
            vd = d_[pl.ds(gi * 16, 16)]
            vs = s_[pl.ds(gi * 16, 16)]
            m = (vd >= lo) & (vd < hi)
            csum = plsc.cumsum(m.astype(jnp.int32))
            pos = cnt + csum - 1
            plsc.store_scatter(slist_v, [pos], vs, mask=m)
            plsc.store_scatter(dlist_v, [pos], (vd - lo) * HID, mask=m)
            plsc.addupdate_scatter(deg_v, [vd - lo], ones16f, mask=m)
            return cnt + csum[15]

        return lax.fori_loop(0, CHUNK // 16, grp, cnt)

    ncp = N_EDGES // CHUNK
    issue(0, 0)
    cnt = jnp.int32(0)
    for c in range(ncp):
        if c + 1 < ncp:
            issue(c + 1, (c + 1) % 2)
        wait(c % 2)
        cnt = scan_chunk(c % 2, cnt)

    cntv_v[...] = jnp.full((16,), cnt, jnp.int32)
    pltpu.sync_copy(slist_v, slist_hbm.at[wid])
    pltpu.sync_copy(dlist_v, dlist_hbm.at[wid])
    pltpu.sync_copy(cntv_v, cnt_hbm.at[wid])
    pltpu.sync_copy(deg_v, deg_hbm.at[wid])


_SC_PARAMS = pltpu.CompilerParams(needs_layout_passes=False)

_preproc = functools.partial(
    pl.kernel,
    compiler_params=_SC_PARAMS,
    out_type=[
        jax.ShapeDtypeStruct((NT, CAP), jnp.int32),
        jax.ShapeDtypeStruct((NT, CAP), jnp.int32),
        jax.ShapeDtypeStruct((NT, 16), jnp.int32),
        jax.ShapeDtypeStruct((NT, NPT), jnp.float32),
    ],
    mesh=_mesh,
    scratch_types=[
        pltpu.VMEM((CHUNK,), jnp.int32),
        pltpu.VMEM((CHUNK,), jnp.int32),
        pltpu.VMEM((CHUNK,), jnp.int32),
        pltpu.VMEM((CHUNK,), jnp.int32),
        pltpu.VMEM((CAP,), jnp.int32),
        pltpu.VMEM((CAP,), jnp.int32),
        pltpu.VMEM((NPT,), jnp.float32),
        pltpu.VMEM((16,), jnp.int32),
        pltpu.SemaphoreType.DMA,
        pltpu.SemaphoreType.DMA,
    ],
)(_preproc_body)


def _make_accum(mode):
    def body(slist_hbm, dlist_hbm, cnt_hbm, g_hbm, dinv_hbm, b_hbm, out_hbm,
             slist_v, dlist_v, rows_a, rows_b, acc_v, dinv_v, b_v, cntc_v,
             psum_v, sem_a, sem_b):
        wid = _wid()
        lo = wid * NPT
        zeros16f = jnp.zeros((16,), jnp.float32)

        pltpu.async_copy(slist_hbm.at[wid], slist_v.at[pl.ds(0, CAP)],
                         sem_a).wait()
        pltpu.async_copy(dlist_hbm.at[wid], dlist_v.at[pl.ds(0, CAP)],
                         sem_a).wait()
        pltpu.async_copy(cnt_hbm.at[wid], cntc_v, sem_a).wait()
        pltpu.async_copy(dinv_hbm.at[pl.ds(lo, NPT)], dinv_v.at[pl.ds(0, NPT)],
                         sem_a).wait()
        pltpu.async_copy(b_hbm, b_v, sem_a).wait()
        cnt = cntc_v[...][0]

        def zero_acc(r, _):
            acc_v[pl.ds(r * 16, 16)] = zeros16f
            return 0
        lax.fori_loop(0, NPT * HID // 16, zero_acc, 0)

        nch = (cnt + G - 1) // G
        bufs = ((rows_a, sem_a), (rows_b, sem_b))

        def issue(c, buf, sm):
            pltpu.async_copy(g_hbm.at[slist_v.at[pl.ds(c * G, G)]], buf, sm)

        def wait(buf, sm):
            pltpu.make_async_copy(g_hbm.at[pl.ds(0, G)], buf, sm).wait()

        def process(c, buf):
            cbase = c * G
            en = jnp.minimum(G, cnt - cbase)

            @plsc.parallel_loop(0, en, unroll=4)
            def _(e):
                dbase = dlist_v[pl.ds(cbase + e, 16)][0]
                for f in range(HID // 16):
                    plsc.addupdate(acc_v.at[pl.ds(dbase + f * 16, 16)],
                                   buf[e, pl.ds(f * 16, 16)])

        @pl.when(nch > 0)
        def _():
            issue(0, rows_a, sem_a)

        def cbody(c, _):
            even = lax.rem(c, 2) == 0

            @pl.when(even)
            def _():
                @pl.when(c + 1 < nch)
                def _():
                    issue(c + 1, rows_b, sem_b)
                wait(rows_a, sem_a)
                process(c, rows_a)

            @pl.when(jnp.logical_not(even))
            def _():
                @pl.when(c + 1 < nch)
                def _():
                    issue(c + 1, rows_a, sem_a)
                wait(rows_b, sem_b)
                process(c, rows_b)

            return 0

        lax.fori_loop(0, nch, cbody, 0)

        pltpu.async_copy(g_hbm.at[pl.ds(lo, NPT)], rows_a.at[pl.ds(0, NPT)],
                         sem_a).wait()

        if mode == "rows":
            def fin(r, _):
                dv = dinv_v[pl.ds(r, 16)][0]
                for f in range(HID // 16):
                    sl = pl.ds(r * HID + f * 16, 16)
                    val = (acc_v[sl] + rows_a[r, pl.ds(f * 16, 16)]) * dv \
                        + b_v[pl.ds(f * 16, 16)]
                    acc_v[sl] = jnp.maximum(val, 0.0)
                return 0
            lax.fori_loop(0, NPT, fin, 0)
            pltpu.sync_copy(acc_v, out_hbm.at[wid])
        else:
            def fin(r, ps):
                dv = dinv_v[pl.ds(r, 16)][0]
                w = jnp.where(lo + r < N_NODES, 1.0, 0.0)
                out = []
                for f in range(HID // 16):
                    val = (acc_v[pl.ds(r * HID + f * 16, 16)]
                           + rows_a[r, pl.ds(f * 16, 16)]) * dv \
                        + b_v[pl.ds(f * 16, 16)]
                    out.append(ps[f] + jnp.maximum(val, 0.0) * w)
                return tuple(out)
            ps = lax.fori_loop(0, NPT, fin,
                               tuple(zeros16f for _ in range(HID // 16)))
            for f in range(HID // 16):
                psum_v[pl.ds(f * 16, 16)] = ps[f]
            for f in range(HID // 16, IN_CH // 16):
                psum_v[pl.ds(f * 16, 16)] = zeros16f
            pltpu.sync_copy(psum_v, out_hbm.at[wid])

    out_shape = (jax.ShapeDtypeStruct((NT, NPT * HID), jnp.float32)
                 if mode == "rows"
                 else jax.ShapeDtypeStruct((NT, IN_CH), jnp.float32))
    return functools.partial(
        pl.kernel,
        compiler_params=_SC_PARAMS,
        out_type=out_shape,
        mesh=_mesh,
        scratch_types=[
            pltpu.VMEM((CAP,), jnp.int32),
            pltpu.VMEM((CAP + 16,), jnp.int32),
            pltpu.VMEM((G, IN_CH), jnp.float32),
            pltpu.VMEM((G, IN_CH), jnp.float32),
            pltpu.VMEM((NPT * HID,), jnp.float32),
            pltpu.VMEM((NPT + 16,), jnp.float32),
            pltpu.VMEM((HID,), jnp.float32),
            pltpu.VMEM((16,), jnp.int32),
            pltpu.VMEM((IN_CH,), jnp.float32),
            pltpu.SemaphoreType.DMA,
            pltpu.SemaphoreType.DMA,
        ],
    )(body)


_accum_rows = _make_accum("rows")
_accum_pool = _make_accum("pool")


_LIN_R = 512


def _lin1_body(x_ref, w_ref, deg_ref, g_ref, dinv_ref):
    dinv = deg_ref[...] ** -0.5
    h = jnp.dot(x_ref[...], w_ref[...], preferred_element_type=jnp.float32)
    g_ref[...] = jnp.concatenate(
        [h * dinv, jnp.zeros((_LIN_R, IN_CH - HID), jnp.float32)], axis=1)
    dinv_ref[...] = dinv


def _lin1(x_pad, w, deg):
    grid = NPAD // _LIN_R
    return pl.pallas_call(
        _lin1_body,
        grid=(grid,),
        in_specs=[
            pl.BlockSpec((_LIN_R, IN_CH), lambda i: (i, 0)),
            pl.BlockSpec((IN_CH, HID), lambda i: (0, 0)),
            pl.BlockSpec((_LIN_R, 1), lambda i: (i, 0)),
        ],
        out_specs=[
            pl.BlockSpec((_LIN_R, IN_CH), lambda i: (i, 0)),
            pl.BlockSpec((_LIN_R, 1), lambda i: (i, 0)),
        ],
        out_shape=[
            jax.ShapeDtypeStruct((NPAD, IN_CH), jnp.float32),
            jax.ShapeDtypeStruct((NPAD, 1), jnp.float32),
        ],
    )(x_pad, w, deg.reshape(NPAD, 1))


def _lin2_body(x_ref, w_ref, dinv_ref, g_ref):
    h = jnp.dot(x_ref[...], w_ref[...], preferred_element_type=jnp.float32)
    g_ref[...] = jnp.concatenate(
        [h * dinv_ref[...], jnp.zeros((_LIN_R, IN_CH - HID), jnp.float32)],
        axis=1)


def _lin2(h, w, dinv):
    grid = NPAD // _LIN_R
    return pl.pallas_call(
        _lin2_body,
        grid=(grid,),
        in_specs=[
            pl.BlockSpec((_LIN_R, HID), lambda i: (i, 0)),
            pl.BlockSpec((HID, HID), lambda i: (0, 0)),
            pl.BlockSpec((_LIN_R, 1), lambda i: (i, 0)),
        ],
        out_specs=pl.BlockSpec((_LIN_R, IN_CH), lambda i: (i, 0)),
        out_shape=jax.ShapeDtypeStruct((NPAD, IN_CH), jnp.float32),
    )(h, w, dinv)


def _head_body(part_ref, muw_ref, mub_ref, lvw_ref, lvb_ref,
               d1w_ref, d1b_ref, eps_ref, mu_ref, lv_ref, d_ref):
    pooled = jnp.sum(part_ref[...], axis=0,
                     keepdims=True)[:, :HID] * (1.0 / N_NODES)
    mu = jnp.dot(pooled, muw_ref[...],
                 preferred_element_type=jnp.float32) + mub_ref[...]
    logvar = jnp.dot(pooled, lvw_ref[...],
                     preferred_element_type=jnp.float32) + lvb_ref[...]
    std = jnp.exp(0.5 * logvar)
    z = mu + eps_ref[...] * std
    d = jnp.dot(z, d1w_ref[...],
                preferred_element_type=jnp.float32) + d1b_ref[...]
    mu_ref[...] = mu
    lv_ref[...] = logvar
    d_ref[...] = jnp.maximum(d, 0.0)


def _head(partials, mu_w, mu_b, lv_w, lv_b, dec1_w, dec1_b, eps):
    return pl.pallas_call(
        _head_body,
        out_shape=[
            jax.ShapeDtypeStruct((1, LAT), jnp.float32),
            jax.ShapeDtypeStruct((1, LAT), jnp.float32),
            jax.ShapeDtypeStruct((1, HID), jnp.float32),
        ],
    )(partials, mu_w, mu_b.reshape(1, LAT), lv_w, lv_b.reshape(1, LAT),
      dec1_w, dec1_b.reshape(1, HID), eps.reshape(1, LAT))


def _dec2_body(d_ref, w_ref, b_ref, out_ref):
    out_ref[...] = jnp.dot(d_ref[...], w_ref[...],
                           preferred_element_type=jnp.float32) + b_ref[...]


def _dec2_matvec(d, dec2_w, dec2_b):
    out = pl.pallas_call(
        _dec2_body,
        grid=(DEC_GRID,),
        in_specs=[
            pl.BlockSpec((1, HID), lambda i: (0, 0)),
            pl.BlockSpec((HID, DEC_BLK), lambda i: (0, i)),
            pl.BlockSpec((1, DEC_BLK), lambda i: (0, i)),
        ],
        out_specs=pl.BlockSpec((1, DEC_BLK), lambda i: (0, i)),
        out_shape=jax.ShapeDtypeStruct((1, DEC_N), jnp.float32),
    )(d, dec2_w, dec2_b.reshape(1, DEC_N))
    return out.reshape(N_NODES, IN_CH)


def kernel(x, edge_index, conv1_w, conv1_b, conv2_w, conv2_b,
           mu_w, mu_b, lv_w, lv_b, dec1_w, dec1_b, dec2_w, dec2_b):
    src = edge_index[0].astype(jnp.int32)
    dst = edge_index[1].astype(jnp.int32)
    x_pad = jnp.pad(x, ((0, NPAD - N_NODES), (0, 0)))
    eps = jax.random.normal(jax.random.key(42), (LAT,), dtype=jnp.float32)

    slist, dlist, cnt, deg = _preproc(src, dst)

    g1, dinv = _lin1(x_pad, conv1_w, deg.reshape(NPAD))
    out1 = _accum_rows(slist, dlist, cnt, g1, dinv.reshape(NPAD), conv1_b)
    g2 = _lin2(out1.reshape(NPAD, HID), conv2_w, dinv)
    partials = _accum_pool(slist, dlist, cnt, g2, dinv.reshape(NPAD), conv2_b)

    mu, logvar, d = _head(partials, mu_w, mu_b, lv_w, lv_b,
                          dec1_w, dec1_b, eps)
    recon_x = _dec2_matvec(d, dec2_w, dec2_b)
    return (recon_x, mu.reshape(LAT), logvar.reshape(LAT))


# speedup vs baseline: 1.0007x; 1.0007x over previous
import functools

import jax
import jax.numpy as jnp
from jax import lax
from jax.experimental import pallas as pl
from jax.experimental.pallas import tpu as pltpu
from jax.experimental.pallas import tpu_sc as plsc

N_NODES = 10000
IN_CH = 128
HID = 64
LAT = 64
N_EDGES = 320000

NC, NS = 2, 16
NT = NC * NS
NPT = 320
NPAD = NT * NPT
CAP = 12288
NSEG = 4
CAPS = CAP // NSEG
CHUNK = 16000
G = 320

DEC_N = N_NODES * IN_CH
DEC_BLK = 25600
DEC_GRID = DEC_N // DEC_BLK

_mesh = plsc.VectorSubcoreMesh(core_axis_name="c", subcore_axis_name="s")


def _wid():
    return lax.axis_index("s") * NC + lax.axis_index("c")


def _preproc_body(src_hbm, dst_hbm, slist_hbm, dlist_hbm, cnt_hbm, deg_hbm,
                  csrc_a, cdst_a, csrc_b, cdst_b, slist_v, dlist_v, deg_v,
                  cntv_v, sem_a, sem_b):
    wid = _wid()
    lo = wid * NPT
    hi = lo + NPT
    zeros16i = jnp.zeros((16,), jnp.int32)
    ones16f = jnp.ones((16,), jnp.float32)

    def init_deg(i, _):
        deg_v[pl.ds(i * 16, 16)] = ones16f
        return 0
    lax.fori_loop(0, NPT // 16, init_deg, 0)

    def init_lists(i, _):
        slist_v[pl.ds(i * 16, 16)] = zeros16i
        return 0
    lax.fori_loop(0, CAP // 16, init_lists, 0)

    bufs = ((csrc_a, cdst_a, sem_a), (csrc_b, cdst_b, sem_b))

    def issue(c, b):
        s_, d_, sm = bufs[b]
        pltpu.async_copy(src_hbm.at[pl.ds(c * CHUNK, CHUNK)], s_, sm)
        pltpu.async_copy(dst_hbm.at[pl.ds(c * CHUNK, CHUNK)], d_, sm)

    def wait(b):
        s_, d_, sm = bufs[b]
        pltpu.make_async_copy(src_hbm.at[pl.ds(0, CHUNK)], s_, sm).wait()
        pltpu.make_async_copy(src_hbm.at[pl.ds(0, CHUNK)], d_, sm).wait()

    def scan_chunk(b, cnt):
        s_, d_, sm = bufs[b]

        def grp(gi, cnt):
            vd = d_[pl.ds(gi * 16, 16)]
            vs = s_[pl.ds(gi * 16, 16)]
            m = (vd >= lo) & (vd < hi)
            csum = plsc.cumsum(m.astype(jnp.int32))
            pos = cnt + csum - 1
            plsc.store_scatter(slist_v, [pos], vs, mask=m)
            plsc.store_scatter(dlist_v, [pos], (vd - lo) * HID, mask=m)
            plsc.addupdate_scatter(deg_v, [vd - lo], ones16f, mask=m)
            return cnt + csum[15]

        return lax.fori_loop(0, CHUNK // 16, grp, cnt)

    ncp = N_EDGES // CHUNK
    issue(0, 0)
    cnt = jnp.int32(0)
    for c in range(ncp):
        if c + 1 < ncp:
            issue(c + 1, (c + 1) % 2)
        wait(c % 2)
        cnt = scan_chunk(c % 2, cnt)

    cntv_v[...] = jnp.full((16,), cnt, jnp.int32)
    pltpu.sync_copy(slist_v, slist_hbm.at[wid])
    pltpu.sync_copy(dlist_v, dlist_hbm.at[wid])
    pltpu.sync_copy(cntv_v, cnt_hbm.at[wid])
    pltpu.sync_copy(deg_v, deg_hbm.at[wid])


_SC_PARAMS = pltpu.CompilerParams(needs_layout_passes=False)

_preproc = functools.partial(
    pl.kernel,
    compiler_params=_SC_PARAMS,
    out_type=[
        jax.ShapeDtypeStruct((NT, CAP), jnp.int32),
        jax.ShapeDtypeStruct((NT, CAP), jnp.int32),
        jax.ShapeDtypeStruct((NT, 16), jnp.int32),
        jax.ShapeDtypeStruct((NT, NPT), jnp.float32),
    ],
    mesh=_mesh,
    scratch_types=[
        pltpu.VMEM((CHUNK,), jnp.int32),
        pltpu.VMEM((CHUNK,), jnp.int32),
        pltpu.VMEM((CHUNK,), jnp.int32),
        pltpu.VMEM((CHUNK,), jnp.int32),
        pltpu.VMEM((CAP,), jnp.int32),
        pltpu.VMEM((CAP,), jnp.int32),
        pltpu.VMEM((NPT,), jnp.float32),
        pltpu.VMEM((16,), jnp.int32),
        pltpu.SemaphoreType.DMA,
        pltpu.SemaphoreType.DMA,
    ],
)(_preproc_body)


def _make_accum(mode):
    def body(slist_hbm, dlist_hbm, cnt_hbm, g_hbm, dinv_hbm, b_hbm, out_hbm,
             slist_v, dlist_v, rows_a, rows_b, acc_v, dinv_v, b_v, cntc_v,
             psum_v, sem_a, sem_b):
        wid = _wid()
        lo = wid * NPT
        zeros16f = jnp.zeros((16,), jnp.float32)

        pltpu.async_copy(slist_hbm.at[wid], slist_v.at[pl.ds(0, CAP)],
                         sem_a).wait()
        pltpu.async_copy(dlist_hbm.at[wid], dlist_v.at[pl.ds(0, CAP)],
                         sem_a).wait()
        pltpu.async_copy(cnt_hbm.at[wid], cntc_v, sem_a).wait()
        pltpu.async_copy(dinv_hbm.at[pl.ds(lo, NPT)], dinv_v.at[pl.ds(0, NPT)],
                         sem_a).wait()
        pltpu.async_copy(b_hbm, b_v, sem_a).wait()
        cnt = cntc_v[...][0]

        def zero_acc(r, _):
            acc_v[pl.ds(r * 16, 16)] = zeros16f
            return 0
        lax.fori_loop(0, NPT * HID // 16, zero_acc, 0)

        nch = (cnt + G - 1) // G
        bufs = ((rows_a, sem_a), (rows_b, sem_b))

        def issue(c, buf, sm):
            pltpu.async_copy(g_hbm.at[slist_v.at[pl.ds(c * G, G)]], buf, sm)

        def wait(buf, sm):
            pltpu.make_async_copy(g_hbm.at[pl.ds(0, G)], buf, sm).wait()

        def process(c, buf):
            cbase = c * G
            en = jnp.minimum(G, cnt - cbase)

            @plsc.parallel_loop(0, en, unroll=4)
            def _(e):
                dbase = dlist_v[pl.ds(cbase + e, 16)][0]
                for f in range(HID // 16):
                    plsc.addupdate(acc_v.at[pl.ds(dbase + f * 16, 16)],
                                   buf[e, pl.ds(f * 16, 16)])

        @pl.when(nch > 0)
        def _():
            issue(0, rows_a, sem_a)

        def cbody(c, _):
            even = lax.rem(c, 2) == 0

            @pl.when(even)
            def _():
                @pl.when(c + 1 < nch)
                def _():
                    issue(c + 1, rows_b, sem_b)
                wait(rows_a, sem_a)
                process(c, rows_a)

            @pl.when(jnp.logical_not(even))
            def _():
                @pl.when(c + 1 < nch)
                def _():
                    issue(c + 1, rows_a, sem_a)
                wait(rows_b, sem_b)
                process(c, rows_b)

            return 0

        lax.fori_loop(0, nch, cbody, 0)

        pltpu.async_copy(g_hbm.at[pl.ds(lo, NPT)], rows_a.at[pl.ds(0, NPT)],
                         sem_a).wait()

        if mode == "rows":
            def fin(r, _):
                dv = dinv_v[pl.ds(r, 16)][0]
                for f in range(HID // 16):
                    sl = pl.ds(r * HID + f * 16, 16)
                    val = (acc_v[sl] + rows_a[r, pl.ds(f * 16, 16)]) * dv \
                        + b_v[pl.ds(f * 16, 16)]
                    acc_v[sl] = jnp.maximum(val, 0.0)
                return 0
            lax.fori_loop(0, NPT, fin, 0)
            pltpu.sync_copy(acc_v, out_hbm.at[wid])
        else:
            def fin(r, ps):
                dv = dinv_v[pl.ds(r, 16)][0]
                w = jnp.where(lo + r < N_NODES, 1.0, 0.0)
                out = []
                for f in range(HID // 16):
                    val = (acc_v[pl.ds(r * HID + f * 16, 16)]
                           + rows_a[r, pl.ds(f * 16, 16)]) * dv \
                        + b_v[pl.ds(f * 16, 16)]
                    out.append(ps[f] + jnp.maximum(val, 0.0) * w)
                return tuple(out)
            ps = lax.fori_loop(0, NPT, fin,
                               tuple(zeros16f for _ in range(HID // 16)))
            for f in range(HID // 16):
                psum_v[pl.ds(f * 16, 16)] = ps[f]
            for f in range(HID // 16, IN_CH // 16):
                psum_v[pl.ds(f * 16, 16)] = zeros16f
            pltpu.sync_copy(psum_v, out_hbm.at[wid])

    out_shape = (jax.ShapeDtypeStruct((NT, NPT * HID), jnp.float32)
                 if mode == "rows"
                 else jax.ShapeDtypeStruct((NT, IN_CH), jnp.float32))
    return functools.partial(
        pl.kernel,
        compiler_params=_SC_PARAMS,
        out_type=out_shape,
        mesh=_mesh,
        scratch_types=[
            pltpu.VMEM((CAP,), jnp.int32),
            pltpu.VMEM((CAP + 16,), jnp.int32),
            pltpu.VMEM((G, IN_CH), jnp.float32),
            pltpu.VMEM((G, IN_CH), jnp.float32),
            pltpu.VMEM((NPT * HID,), jnp.float32),
            pltpu.VMEM((NPT + 16,), jnp.float32),
            pltpu.VMEM((HID,), jnp.float32),
            pltpu.VMEM((16,), jnp.int32),
            pltpu.VMEM((IN_CH,), jnp.float32),
            pltpu.SemaphoreType.DMA,
            pltpu.SemaphoreType.DMA,
        ],
    )(body)


_accum_rows = _make_accum("rows")
_accum_pool = _make_accum("pool")


_LIN_R = 512


def _lin1_body(x_ref, w_ref, deg_ref, g_ref, dinv_ref):
    dinv = deg_ref[...] ** -0.5
    h = jnp.dot(x_ref[...], w_ref[...], preferred_element_type=jnp.float32)
    g_ref[...] = jnp.concatenate(
        [h * dinv, jnp.zeros((_LIN_R, IN_CH - HID), jnp.float32)], axis=1)
    dinv_ref[...] = dinv


def _lin1(x_pad, w, deg):
    grid = NPAD // _LIN_R
    return pl.pallas_call(
        _lin1_body,
        grid=(grid,),
        in_specs=[
            pl.BlockSpec((_LIN_R, IN_CH), lambda i: (i, 0)),
            pl.BlockSpec((IN_CH, HID), lambda i: (0, 0)),
            pl.BlockSpec((_LIN_R, 1), lambda i: (i, 0)),
        ],
        out_specs=[
            pl.BlockSpec((_LIN_R, IN_CH), lambda i: (i, 0)),
            pl.BlockSpec((_LIN_R, 1), lambda i: (i, 0)),
        ],
        out_shape=[
            jax.ShapeDtypeStruct((NPAD, IN_CH), jnp.float32),
            jax.ShapeDtypeStruct((NPAD, 1), jnp.float32),
        ],
    )(x_pad, w, deg.reshape(NPAD, 1))


def _lin2_body(x_ref, w_ref, dinv_ref, g_ref):
    h = jnp.dot(x_ref[...], w_ref[...], preferred_element_type=jnp.float32)
    g_ref[...] = jnp.concatenate(
        [h * dinv_ref[...], jnp.zeros((_LIN_R, IN_CH - HID), jnp.float32)],
        axis=1)


def _lin2(h, w, dinv):
    grid = NPAD // _LIN_R
    return pl.pallas_call(
        _lin2_body,
        grid=(grid,),
        in_specs=[
            pl.BlockSpec((_LIN_R, HID), lambda i: (i, 0)),
            pl.BlockSpec((HID, HID), lambda i: (0, 0)),
            pl.BlockSpec((_LIN_R, 1), lambda i: (i, 0)),
        ],
        out_specs=pl.BlockSpec((_LIN_R, IN_CH), lambda i: (i, 0)),
        out_shape=jax.ShapeDtypeStruct((NPAD, IN_CH), jnp.float32),
    )(h, w, dinv)


def _head_body(part_ref, muw_ref, mub_ref, lvw_ref, lvb_ref,
               d1w_ref, d1b_ref, eps_ref, mu_ref, lv_ref, d_ref):
    pooled = jnp.sum(part_ref[...], axis=0,
                     keepdims=True)[:, :HID] * (1.0 / N_NODES)
    mu = jnp.dot(pooled, muw_ref[...],
                 preferred_element_type=jnp.float32) + mub_ref[...]
    logvar = jnp.dot(pooled, lvw_ref[...],
                     preferred_element_type=jnp.float32) + lvb_ref[...]
    std = jnp.exp(0.5 * logvar)
    z = mu + eps_ref[...] * std
    d = jnp.dot(z, d1w_ref[...],
                preferred_element_type=jnp.float32) + d1b_ref[...]
    mu_ref[...] = mu
    lv_ref[...] = logvar
    d_ref[...] = jnp.maximum(d, 0.0)


def _head(partials, mu_w, mu_b, lv_w, lv_b, dec1_w, dec1_b, eps):
    return pl.pallas_call(
        _head_body,
        out_shape=[
            jax.ShapeDtypeStruct((1, LAT), jnp.float32),
            jax.ShapeDtypeStruct((1, LAT), jnp.float32),
            jax.ShapeDtypeStruct((1, HID), jnp.float32),
        ],
    )(partials, mu_w, mu_b.reshape(1, LAT), lv_w, lv_b.reshape(1, LAT),
      dec1_w, dec1_b.reshape(1, HID), eps.reshape(1, LAT))


def _dec2_body(d_ref, w_ref, b_ref, out_ref):
    out_ref[...] = jnp.dot(d_ref[...], w_ref[...],
                           preferred_element_type=jnp.float32) + b_ref[...]


def _dec2_matvec(d, dec2_w, dec2_b):
    out = pl.pallas_call(
        _dec2_body,
        grid=(DEC_GRID,),
        in_specs=[
            pl.BlockSpec((1, HID), lambda i: (0, 0)),
            pl.BlockSpec((HID, DEC_BLK), lambda i: (0, i)),
            pl.BlockSpec((1, DEC_BLK), lambda i: (0, i)),
        ],
        out_specs=pl.BlockSpec((1, DEC_BLK), lambda i: (0, i)),
        out_shape=jax.ShapeDtypeStruct((1, DEC_N), jnp.float32),
    )(d, dec2_w, dec2_b.reshape(1, DEC_N))
    return out.reshape(N_NODES, IN_CH)


def kernel(x, edge_index, conv1_w, conv1_b, conv2_w, conv2_b,
           mu_w, mu_b, lv_w, lv_b, dec1_w, dec1_b, dec2_w, dec2_b):
    src = edge_index[0].astype(jnp.int32)
    dst = edge_index[1].astype(jnp.int32)
    x_pad = jnp.pad(x, ((0, NPAD - N_NODES), (0, 0)))
    eps = jax.random.normal(jax.random.key(42), (LAT,), dtype=jnp.float32)

    slist, dlist, cnt, deg = _preproc(src, dst)

    g1, dinv = _lin1(x_pad, conv1_w, deg.reshape(NPAD))
    out1 = _accum_rows(slist, dlist, cnt, g1, dinv.reshape(NPAD), conv1_b)
    g2 = _lin2(out1.reshape(NPAD, HID), conv2_w, dinv)
    partials = _accum_pool(slist, dlist, cnt, g2, dinv.reshape(NPAD), conv2_b)

    mu, logvar, d = _head(partials, mu_w, mu_b, lv_w, lv_b,
                          dec1_w, dec1_b, eps)
    recon_x = _dec2_matvec(d, dec2_w, dec2_b)
    return (recon_x, mu.reshape(LAT), logvar.reshape(LAT))
